# baseline (speedup 1.0000x reference)
import functools

import jax
import jax.numpy as jnp
from jax.experimental import pallas as pl

_PREC = jax.lax.Precision.HIGHEST


def _dot(a, b):
    return jnp.dot(a, b, precision=_PREC, preferred_element_type=jnp.float32)


def _dot_t(a, b):
    return jax.lax.dot_general(a, b, (((1,), (1,)), ((), ())),
                               precision=_PREC,
                               preferred_element_type=jnp.float32)


def _obj_pn_kernel(x_ref, wt1, bt1, wt2, bt2, wt3, bt3,
                   wp1, bp1, wp2, bp2, wp3, bp3, ot_ref, op_ref):
    c = pl.program_id(1)
    x = x_ref[0]

    def chain(w1, b1, w2, b2, w3, b3):
        h = jnp.maximum(_dot(x, w1[...]) + b1[...], 0.0)
        h = jnp.maximum(_dot(h, w2[...]) + b2[...], 0.0)
        h = _dot(h, w3[...]) + b3[...]
        return jnp.max(h, axis=0, keepdims=True)

    pt = chain(wt1, bt1, wt2, bt2, wt3, bt3)[None]
    pp = chain(wp1, bp1, wp2, bp2, wp3, bp3)[None]

    @pl.when(c == 0)
    def _():
        ot_ref[...] = pt
        op_ref[...] = pp

    @pl.when(c != 0)
    def _():
        ot_ref[...] = jnp.maximum(ot_ref[...], pt)
        op_ref[...] = jnp.maximum(op_ref[...], pp)


def _obj_pointnets(objx, pt_t, pt_p):
    B, N, C = objx.shape
    P = 1024
    nc = N // P
    full = lambda s: pl.BlockSpec(s, lambda b, c: (0,) * len(s))
    wspecs = []
    args = []
    for p in (pt_t, pt_p):
        for k in ('w1', 'b1', 'w2', 'b2', 'w3', 'b3'):
            a = p[k]
            if a.ndim == 1:
                a = a.reshape(1, -1)
            args.append(a)
            wspecs.append(full(a.shape))
    ot, op = pl.pallas_call(
        _obj_pn_kernel,
        grid=(B, nc),
        in_specs=[pl.BlockSpec((1, P, C), lambda b, c: (b, c, 0))] + wspecs,
        out_specs=[pl.BlockSpec((1, 1, 1024), lambda b, c: (b, 0, 0)),
                   pl.BlockSpec((1, 1, 1024), lambda b, c: (b, 0, 0))],
        out_shape=[jax.ShapeDtypeStruct((B, 1, 1024), jnp.float32),
                   jax.ShapeDtypeStruct((B, 1, 1024), jnp.float32)],
    )(objx, *args)
    return ot.reshape(B, 1024), op.reshape(B, 1024)


_FJ = [0, 1, 2, 3, 4, 5, 5]
_STARTS = [0, 83, 206, 326, 448, 569, 697]
_VALID = [83, 123, 120, 122, 121, 128, 81]


def _hand_pn_kernel(chunk_ref, nat_ref, w1, b1, w2, b2, w3, b3, out_ref):
    j = pl.program_id(0)
    v = jnp.int32(_VALID[-1])
    for jj in range(6):
        v = jnp.where(j == jj, jnp.int32(_VALID[jj]), v)
    nat = nat_ref[...]
    mean = jnp.sum(nat, axis=2) * (1.0 / 778.0)
    B = nat.shape[0]
    x = chunk_ref[0].reshape(B, 128, 3) - mean[:, None, :]
    x = x.reshape(B * 128, 3)
    h = jnp.maximum(_dot(x, w1[0]) + b1[0], 0.0)
    h = jnp.maximum(_dot(h, w2[0]) + b2[0], 0.0)
    h = _dot(h, w3[0]) + b3[0]
    h = h.reshape(B, 128, 1024)
    pid = jax.lax.broadcasted_iota(jnp.int32, (B, 128, 1), 1)
    h = jnp.where(pid < v, h, -1e30)
    pm = jnp.max(h, axis=1)

    @pl.when(j < 6)
    def _():
        out_ref[0] = pm

    @pl.when(j == 6)
    def _():
        out_ref[0] = jnp.maximum(out_ref[0], pm)


def _hand_pointnets(hand_xyz, enc):
    B = hand_xyz.shape[0]
    hp = jnp.transpose(hand_xyz, (0, 2, 1))
    hp = jnp.pad(hp, ((0, 0), (0, 832 - 778), (0, 0)))
    chunks = jnp.stack([hp[:, s:s + 128, :] for s in _STARTS])
    chunks = chunks.reshape(7, B * 128, 3)
    nat = jnp.pad(hand_xyz, ((0, 0), (0, 0), (0, 832 - 778)))

    stk = lambda k: jnp.stack([enc[i][k] for i in range(6)])
    W1, W2, W3 = stk('w1'), stk('w2'), stk('w3')
    B1 = stk('b1')[:, None, :]
    B2 = stk('b2')[:, None, :]
    B3 = stk('b3')[:, None, :]

    wmap = lambda j: (jnp.minimum(j, 5), 0, 0)
    return pl.pallas_call(
        _hand_pn_kernel,
        grid=(7,),
        in_specs=[
            pl.BlockSpec((1, B * 128, 3), lambda j: (j, 0, 0)),
            pl.BlockSpec(nat.shape, lambda j: (0, 0, 0)),
            pl.BlockSpec((1,) + W1.shape[1:], wmap),
            pl.BlockSpec((1,) + B1.shape[1:], wmap),
            pl.BlockSpec((1,) + W2.shape[1:], wmap),
            pl.BlockSpec((1,) + B2.shape[1:], wmap),
            pl.BlockSpec((1,) + W3.shape[1:], wmap),
            pl.BlockSpec((1,) + B3.shape[1:], wmap),
        ],
        out_specs=pl.BlockSpec((1, B, 1024), wmap),
        out_shape=jax.ShapeDtypeStruct((6, B, 1024), jnp.float32),
    )(chunks, nat, W1, B1, W2, B2, W3, B3)


def _emb_vq_kernel(feat_ref, w0, b0, wm, bm, cb_ref, q_ref, ssq_ref):
    f = feat_ref[0]
    h = jnp.maximum(_dot(f, w0[0]) + b0[0], 0.0)
    z = _dot(h, wm[0]) + bm[0]
    cb = cb_ref[0]
    d = (jnp.sum(z * z, axis=1, keepdims=True)
         - 2.0 * _dot_t(z, cb)
         + jnp.sum(cb * cb, axis=1)[None, :])
    B = d.shape[0]
    lane = jax.lax.broadcasted_iota(jnp.int32, (B, 128), 1)
    dmin = jnp.min(d, axis=1, keepdims=True)
    idx = jnp.min(jnp.where(d == dmin, lane, 128), axis=1, keepdims=True)
    onehot = (lane == idx).astype(jnp.float32)
    q = _dot(onehot, cb)
    q_ref[0] = q
    ssq_ref[...] = jnp.sum((q - z) ** 2).reshape(1, 1, 1)


def _emb_vq(feat, emb, cbs):
    B = feat.shape[1]
    stk = lambda k: jnp.stack([emb[i][k] for i in range(6)])
    W0, WM = stk('w0'), stk('wm')
    B0 = stk('b0')[:, None, :]
    BM = stk('bm')[:, None, :]
    CB = jnp.stack(cbs)
    bmap = lambda i: (i, 0, 0)
    return pl.pallas_call(
        _emb_vq_kernel,
        grid=(6,),
        in_specs=[
            pl.BlockSpec((1, B, 1024), bmap),
            pl.BlockSpec((1,) + W0.shape[1:], bmap),
            pl.BlockSpec((1,) + B0.shape[1:], bmap),
            pl.BlockSpec((1,) + WM.shape[1:], bmap),
            pl.BlockSpec((1,) + BM.shape[1:], bmap),
            pl.BlockSpec((1,) + CB.shape[1:], bmap),
        ],
        out_specs=[pl.BlockSpec((1, B, 256), bmap),
                   pl.BlockSpec((1, 1, 1), lambda i: (i, 0, 0))],
        out_shape=[jax.ShapeDtypeStruct((6, B, 256), jnp.float32),
                   jax.ShapeDtypeStruct((6, 1, 1), jnp.float32)],
    )(feat, W0, B0, WM, BM, CB)


def _final_kernel(q_ref, ot_ref, op_ref, cb6_ref, ssq_ref,
                  d0, db0, d1, db1, d2p, db2p,
                  p0, pb0, p1, pb1, p2p, pb2p, out_ref, loss_ref):
    opos = op_ref[...]
    cb6 = cb6_ref[...]
    d = (jnp.sum(opos * opos, axis=1, keepdims=True)
         - 2.0 * _dot_t(opos, cb6)
         + jnp.sum(cb6 * cb6, axis=1)[None, :])
    B = d.shape[0]
    lane = jax.lax.broadcasted_iota(jnp.int32, (B, 128), 1)
    dmin = jnp.min(d, axis=1, keepdims=True)
    idx = jnp.min(jnp.where(d == dmin, lane, 128), axis=1, keepdims=True)
    onehot = (lane == idx).astype(jnp.float32)
    q6 = _dot(onehot, cb6)
    ssq6 = jnp.sum((q6 - opos) ** 2)
    loss = (1.25 * jnp.sum(ssq_ref[...]) / (B * 256.0)
            + 3.0 * ssq6 / (B * 1024.0))

    otype = ot_ref[...]
    x1 = _dot(otype, d0[1536:, :]) + db0[...]
    for i in range(6):
        x1 = x1 + _dot(q_ref[i], d0[256 * i:256 * (i + 1), :])
    h = jnp.maximum(x1, 0.0)
    h = jnp.maximum(_dot(h, d1[...]) + db1[...], 0.0)
    recon = _dot(h, d2p[...]) + db2p[...]
    y1 = _dot(q6, p0[:1024, :]) + _dot(otype, p0[1024:, :]) + pb0[...]
    g = jnp.maximum(y1, 0.0)
    g = jnp.maximum(_dot(g, p1[...]) + pb1[...], 0.0)
    pos = _dot(g, p2p[...]) + pb2p[...]
    out_ref[...] = recon + pos
    loss_ref[...] = loss.reshape(1, 1)


def _final(q, otype, opos, cb6, ssq, dec, pos_dec):
    B = otype.shape[0]
    d2p = jnp.pad(dec['w2'], ((0, 0), (0, 6)))
    db2p = jnp.pad(dec['b2'], (0, 6)).reshape(1, 61)
    p2p = jnp.pad(pos_dec['w2'], ((0, 0), (55, 0)))
    pb2p = jnp.pad(pos_dec['b2'], (55, 0)).reshape(1, 61)
    args = [q, otype, opos, cb6, ssq,
            dec['w0'], dec['b0'].reshape(1, -1),
            dec['w1'], dec['b1'].reshape(1, -1), d2p, db2p,
            pos_dec['w0'], pos_dec['b0'].reshape(1, -1),
            pos_dec['w1'], pos_dec['b1'].reshape(1, -1), p2p, pb2p]
    out, loss = pl.pallas_call(
        _final_kernel,
        in_specs=[pl.BlockSpec(a.shape, functools.partial(lambda n: (0,) * n, a.ndim))
                  for a in args],
        out_specs=[pl.BlockSpec((B, 61), lambda: (0, 0)),
                   pl.BlockSpec((1, 1), lambda: (0, 0))],
        out_shape=[jax.ShapeDtypeStruct((B, 61), jnp.float32),
                   jax.ShapeDtypeStruct((1, 1), jnp.float32)],
    )(*args)
    return out, loss


def kernel(obj_pc, hand_xyz, params):
    objx = jnp.transpose(obj_pc, (0, 2, 1))
    otype, opos = _obj_pointnets(objx, params['obj_type'], params['obj_pos'])
    feat = _hand_pointnets(hand_xyz, params['hand_enc'])
    q, ssq = _emb_vq(feat, params['emb'], params['cb'])
    out, loss = _final(q, otype, opos, params['cb6'], ssq,
                       params['dec'], params['pos_dec'])
    return out, loss[0, 0]


# speedup vs baseline: 4.0018x; 4.0018x over previous
import functools

import jax
import jax.numpy as jnp
from jax.experimental import pallas as pl

def _dot(a, b):
    return jnp.dot(a, b, preferred_element_type=jnp.float32)


_bdot = _dot


def _dot_t(a, b):
    return jax.lax.dot_general(a, b, (((1,), (1,)), ((), ())),
                               preferred_element_type=jnp.float32)


def _obj_pn_kernel(x_ref, wt1, bt1, wt2, bt2, wt3, bt3,
                   wp1, bp1, wp2, bp2, wp3, bp3, ot_ref, op_ref):
    c = pl.program_id(1)
    x = x_ref[0]

    def chain(w1, b1, w2, b2, w3, b3):
        h = jnp.maximum(_bdot(x, w1[...]) + b1[...], 0.0)
        h = jnp.maximum(_bdot(h, w2[...]) + b2[...], 0.0)
        h = _bdot(h, w3[...]) + b3[...]
        return jnp.max(h, axis=0, keepdims=True)

    pt = chain(wt1, bt1, wt2, bt2, wt3, bt3)[None]
    pp = chain(wp1, bp1, wp2, bp2, wp3, bp3)[None]

    @pl.when(c == 0)
    def _():
        ot_ref[...] = pt
        op_ref[...] = pp

    @pl.when(c != 0)
    def _():
        ot_ref[...] = jnp.maximum(ot_ref[...], pt)
        op_ref[...] = jnp.maximum(op_ref[...], pp)


def _obj_pointnets(objx, pt_t, pt_p):
    B, N, C = objx.shape
    P = 1024
    nc = N // P
    full = lambda s: pl.BlockSpec(s, lambda b, c: (0,) * len(s))
    wspecs = []
    args = []
    for p in (pt_t, pt_p):
        for k in ('w1', 'b1', 'w2', 'b2', 'w3', 'b3'):
            a = p[k]
            if a.ndim == 1:
                a = a.reshape(1, -1)
            args.append(a)
            wspecs.append(full(a.shape))
    ot, op = pl.pallas_call(
        _obj_pn_kernel,
        grid=(B, nc),
        in_specs=[pl.BlockSpec((1, P, C), lambda b, c: (b, c, 0))] + wspecs,
        out_specs=[pl.BlockSpec((1, 1, 1024), lambda b, c: (b, 0, 0)),
                   pl.BlockSpec((1, 1, 1024), lambda b, c: (b, 0, 0))],
        out_shape=[jax.ShapeDtypeStruct((B, 1, 1024), jnp.float32),
                   jax.ShapeDtypeStruct((B, 1, 1024), jnp.float32)],
    )(objx, *args)
    return ot.reshape(B, 1024), op.reshape(B, 1024)


_FJ = [0, 1, 2, 3, 4, 5, 5]
_STARTS = [0, 83, 206, 326, 448, 569, 697]
_VALID = [83, 123, 120, 122, 121, 128, 81]


def _hand_pn_kernel(chunk_ref, nat_ref, w1, b1, w2, b2, w3, b3, out_ref):
    j = pl.program_id(0)
    v = jnp.int32(_VALID[-1])
    for jj in range(6):
        v = jnp.where(j == jj, jnp.int32(_VALID[jj]), v)
    nat = nat_ref[...]
    mean = jnp.sum(nat, axis=2) * (1.0 / 778.0)
    B = nat.shape[0]
    x = chunk_ref[0].reshape(B, 128, 3) - mean[:, None, :]
    x = x.reshape(B * 128, 3)
    h = jnp.maximum(_bdot(x, w1[0]) + b1[0], 0.0)
    h = jnp.maximum(_bdot(h, w2[0]) + b2[0], 0.0)
    h = _bdot(h, w3[0]) + b3[0]
    h = h.reshape(B, 128, 1024)
    pid = jax.lax.broadcasted_iota(jnp.int32, (B, 128, 1), 1)
    h = jnp.where(pid < v, h, -1e30)
    pm = jnp.max(h, axis=1)

    @pl.when(j < 6)
    def _():
        out_ref[0] = pm

    @pl.when(j == 6)
    def _():
        out_ref[0] = jnp.maximum(out_ref[0], pm)


def _hand_pointnets(hand_xyz, enc):
    B = hand_xyz.shape[0]
    hp = jnp.transpose(hand_xyz, (0, 2, 1))
    hp = jnp.pad(hp, ((0, 0), (0, 832 - 778), (0, 0)))
    chunks = jnp.stack([hp[:, s:s + 128, :] for s in _STARTS])
    chunks = chunks.reshape(7, B * 128, 3)
    nat = jnp.pad(hand_xyz, ((0, 0), (0, 0), (0, 832 - 778)))

    stk = lambda k: jnp.stack([enc[i][k] for i in range(6)])
    W1, W2, W3 = stk('w1'), stk('w2'), stk('w3')
    B1 = stk('b1')[:, None, :]
    B2 = stk('b2')[:, None, :]
    B3 = stk('b3')[:, None, :]

    wmap = lambda j: (jnp.minimum(j, 5), 0, 0)
    return pl.pallas_call(
        _hand_pn_kernel,
        grid=(7,),
        in_specs=[
            pl.BlockSpec((1, B * 128, 3), lambda j: (j, 0, 0)),
            pl.BlockSpec(nat.shape, lambda j: (0, 0, 0)),
            pl.BlockSpec((1,) + W1.shape[1:], wmap),
            pl.BlockSpec((1,) + B1.shape[1:], wmap),
            pl.BlockSpec((1,) + W2.shape[1:], wmap),
            pl.BlockSpec((1,) + B2.shape[1:], wmap),
            pl.BlockSpec((1,) + W3.shape[1:], wmap),
            pl.BlockSpec((1,) + B3.shape[1:], wmap),
        ],
        out_specs=pl.BlockSpec((1, B, 1024), wmap),
        out_shape=jax.ShapeDtypeStruct((6, B, 1024), jnp.float32),
    )(chunks, nat, W1, B1, W2, B2, W3, B3)


def _emb_vq_kernel(feat_ref, w0, b0, wm, bm, cb_ref, q_ref, ssq_ref):
    f = feat_ref[0]
    h = jnp.maximum(_bdot(f, w0[0]) + b0[0], 0.0)
    z = _bdot(h, wm[0]) + bm[0]
    cb = cb_ref[0]
    d = (jnp.sum(z * z, axis=1, keepdims=True)
         - 2.0 * _dot_t(z, cb)
         + jnp.sum(cb * cb, axis=1)[None, :])
    B = d.shape[0]
    lane = jax.lax.broadcasted_iota(jnp.int32, (B, 128), 1)
    dmin = jnp.min(d, axis=1, keepdims=True)
    idx = jnp.min(jnp.where(d == dmin, lane, 128), axis=1, keepdims=True)
    onehot = (lane == idx).astype(jnp.float32)
    q = _dot(onehot, cb)
    q_ref[0] = q
    ssq_ref[...] = jnp.sum((q - z) ** 2).reshape(1, 1, 1)


def _emb_vq(feat, emb, cbs):
    B = feat.shape[1]
    stk = lambda k: jnp.stack([emb[i][k] for i in range(6)])
    W0, WM = stk('w0'), stk('wm')
    B0 = stk('b0')[:, None, :]
    BM = stk('bm')[:, None, :]
    CB = jnp.stack(cbs)
    bmap = lambda i: (i, 0, 0)
    return pl.pallas_call(
        _emb_vq_kernel,
        grid=(6,),
        in_specs=[
            pl.BlockSpec((1, B, 1024), bmap),
            pl.BlockSpec((1,) + W0.shape[1:], bmap),
            pl.BlockSpec((1,) + B0.shape[1:], bmap),
            pl.BlockSpec((1,) + WM.shape[1:], bmap),
            pl.BlockSpec((1,) + BM.shape[1:], bmap),
            pl.BlockSpec((1,) + CB.shape[1:], bmap),
        ],
        out_specs=[pl.BlockSpec((1, B, 256), bmap),
                   pl.BlockSpec((1, 1, 1), lambda i: (i, 0, 0))],
        out_shape=[jax.ShapeDtypeStruct((6, B, 256), jnp.float32),
                   jax.ShapeDtypeStruct((6, 1, 1), jnp.float32)],
    )(feat, W0, B0, WM, BM, CB)


def _final_kernel(q_ref, ot_ref, op_ref, cb6_ref, ssq_ref,
                  d0, db0, d1, db1, d2p, db2p,
                  p0, pb0, p1, pb1, p2p, pb2p, out_ref, loss_ref):
    opos = op_ref[...]
    cb6 = cb6_ref[...]
    d = (jnp.sum(opos * opos, axis=1, keepdims=True)
         - 2.0 * _dot_t(opos, cb6)
         + jnp.sum(cb6 * cb6, axis=1)[None, :])
    B = d.shape[0]
    lane = jax.lax.broadcasted_iota(jnp.int32, (B, 128), 1)
    dmin = jnp.min(d, axis=1, keepdims=True)
    idx = jnp.min(jnp.where(d == dmin, lane, 128), axis=1, keepdims=True)
    onehot = (lane == idx).astype(jnp.float32)
    q6 = _dot(onehot, cb6)
    ssq6 = jnp.sum((q6 - opos) ** 2)
    loss = (1.25 * jnp.sum(ssq_ref[...]) / (B * 256.0)
            + 3.0 * ssq6 / (B * 1024.0))

    otype = ot_ref[...]
    x1 = _bdot(otype, d0[1536:, :]) + db0[...]
    for i in range(6):
        x1 = x1 + _bdot(q_ref[i], d0[256 * i:256 * (i + 1), :])
    h = jnp.maximum(x1, 0.0)
    h = jnp.maximum(_bdot(h, d1[...]) + db1[...], 0.0)
    recon = _bdot(h, d2p[...]) + db2p[...]
    y1 = _bdot(q6, p0[:1024, :]) + _bdot(otype, p0[1024:, :]) + pb0[...]
    g = jnp.maximum(y1, 0.0)
    g = jnp.maximum(_bdot(g, p1[...]) + pb1[...], 0.0)
    pos = _bdot(g, p2p[...]) + pb2p[...]
    out_ref[...] = recon + pos
    loss_ref[...] = loss.reshape(1, 1)


def _final(q, otype, opos, cb6, ssq, dec, pos_dec):
    B = otype.shape[0]
    d2p = jnp.pad(dec['w2'], ((0, 0), (0, 6)))
    db2p = jnp.pad(dec['b2'], (0, 6)).reshape(1, 61)
    p2p = jnp.pad(pos_dec['w2'], ((0, 0), (55, 0)))
    pb2p = jnp.pad(pos_dec['b2'], (55, 0)).reshape(1, 61)
    args = [q, otype, opos, cb6, ssq,
            dec['w0'], dec['b0'].reshape(1, -1),
            dec['w1'], dec['b1'].reshape(1, -1), d2p, db2p,
            pos_dec['w0'], pos_dec['b0'].reshape(1, -1),
            pos_dec['w1'], pos_dec['b1'].reshape(1, -1), p2p, pb2p]
    out, loss = pl.pallas_call(
        _final_kernel,
        in_specs=[pl.BlockSpec(a.shape, functools.partial(lambda n: (0,) * n, a.ndim))
                  for a in args],
        out_specs=[pl.BlockSpec((B, 61), lambda: (0, 0)),
                   pl.BlockSpec((1, 1), lambda: (0, 0))],
        out_shape=[jax.ShapeDtypeStruct((B, 61), jnp.float32),
                   jax.ShapeDtypeStruct((1, 1), jnp.float32)],
    )(*args)
    return out, loss


def kernel(obj_pc, hand_xyz, params):
    objx = jnp.transpose(obj_pc, (0, 2, 1))
    otype, opos = _obj_pointnets(objx, params['obj_type'], params['obj_pos'])
    feat = _hand_pointnets(hand_xyz, params['hand_enc'])
    q, ssq = _emb_vq(feat, params['emb'], params['cb'])
    out, loss = _final(q, otype, opos, params['cb6'], ssq,
                       params['dec'], params['pos_dec'])
    return out, loss[0, 0]


# speedup vs baseline: 4.5252x; 1.1308x over previous
import functools

import jax
import jax.numpy as jnp
from jax.experimental import pallas as pl

def _dot(a, b):
    return jnp.dot(a, b, preferred_element_type=jnp.float32)


_bdot = _dot


def _dot_t(a, b):
    return jax.lax.dot_general(a, b, (((1,), (1,)), ((), ())),
                               preferred_element_type=jnp.float32)


_BPC = 2
_NPB = 2048


def _obj_pn_kernel(x_ref, wt1, bt1, wt2, bt2, wt3, bt3,
                   wp1, bp1, wp2, bp2, wp3, bp3, ot_ref, op_ref):
    x = x_ref[...]

    def chain(w1, b1, w2, b2, w3, b3):
        h = jnp.maximum(_bdot(x, w1[...]) + b1[...], 0.0)
        h = jnp.maximum(_bdot(h, w2[...]) + b2[...], 0.0)
        h = _bdot(h, w3[...]) + b3[...]
        h = h.reshape(_BPC, _NPB, 1024)
        return jnp.max(h, axis=1)[:, None, :]

    ot_ref[...] = chain(wt1, bt1, wt2, bt2, wt3, bt3)
    op_ref[...] = chain(wp1, bp1, wp2, bp2, wp3, bp3)


def _obj_pointnets(objx, pt_t, pt_p):
    B, N, C = objx.shape
    xflat = objx.reshape(B * N, C)
    steps = B // _BPC
    full = lambda s: pl.BlockSpec(s, lambda c: (0,) * len(s))
    wspecs = []
    args = []
    for p in (pt_t, pt_p):
        for k in ('w1', 'b1', 'w2', 'b2', 'w3', 'b3'):
            a = p[k]
            if a.ndim == 1:
                a = a.reshape(1, -1)
            args.append(a)
            wspecs.append(full(a.shape))
    ot, op = pl.pallas_call(
        _obj_pn_kernel,
        grid=(steps,),
        in_specs=[pl.BlockSpec((_BPC * _NPB, C), lambda c: (c, 0))] + wspecs,
        out_specs=[pl.BlockSpec((_BPC, 1, 1024), lambda c: (c, 0, 0)),
                   pl.BlockSpec((_BPC, 1, 1024), lambda c: (c, 0, 0))],
        out_shape=[jax.ShapeDtypeStruct((B, 1, 1024), jnp.float32),
                   jax.ShapeDtypeStruct((B, 1, 1024), jnp.float32)],
    )(xflat, *args)
    return ot.reshape(B, 1024), op.reshape(B, 1024)


_FJ = [0, 1, 2, 3, 4, 5, 5]
_STARTS = [0, 83, 206, 326, 448, 569, 697]
_VALID = [83, 123, 120, 122, 121, 128, 81]


def _hand_pn_kernel(chunk_ref, nat_ref, w1, b1, w2, b2, w3, b3, out_ref):
    j = pl.program_id(0)
    v = jnp.int32(_VALID[-1])
    for jj in range(6):
        v = jnp.where(j == jj, jnp.int32(_VALID[jj]), v)
    nat = nat_ref[...]
    mean = jnp.sum(nat, axis=2) * (1.0 / 778.0)
    B = nat.shape[0]
    x = chunk_ref[0].reshape(B, 128, 3) - mean[:, None, :]
    x = x.reshape(B * 128, 3)
    h = jnp.maximum(_bdot(x, w1[0]) + b1[0], 0.0)
    h = jnp.maximum(_bdot(h, w2[0]) + b2[0], 0.0)
    h = _bdot(h, w3[0]) + b3[0]
    h = h.reshape(B, 128, 1024)
    pid = jax.lax.broadcasted_iota(jnp.int32, (B, 128, 1), 1)
    h = jnp.where(pid < v, h, -1e30)
    pm = jnp.max(h, axis=1)

    @pl.when(j < 6)
    def _():
        out_ref[0] = pm

    @pl.when(j == 6)
    def _():
        out_ref[0] = jnp.maximum(out_ref[0], pm)


def _hand_pointnets(hand_xyz, enc):
    B = hand_xyz.shape[0]
    hp = jnp.transpose(hand_xyz, (0, 2, 1))
    hp = jnp.pad(hp, ((0, 0), (0, 832 - 778), (0, 0)))
    chunks = jnp.stack([hp[:, s:s + 128, :] for s in _STARTS])
    chunks = chunks.reshape(7, B * 128, 3)
    nat = jnp.pad(hand_xyz, ((0, 0), (0, 0), (0, 832 - 778)))

    stk = lambda k: jnp.stack([enc[i][k] for i in range(6)])
    W1, W2, W3 = stk('w1'), stk('w2'), stk('w3')
    B1 = stk('b1')[:, None, :]
    B2 = stk('b2')[:, None, :]
    B3 = stk('b3')[:, None, :]

    wmap = lambda j: (jnp.minimum(j, 5), 0, 0)
    return pl.pallas_call(
        _hand_pn_kernel,
        grid=(7,),
        in_specs=[
            pl.BlockSpec((1, B * 128, 3), lambda j: (j, 0, 0)),
            pl.BlockSpec(nat.shape, lambda j: (0, 0, 0)),
            pl.BlockSpec((1,) + W1.shape[1:], wmap),
            pl.BlockSpec((1,) + B1.shape[1:], wmap),
            pl.BlockSpec((1,) + W2.shape[1:], wmap),
            pl.BlockSpec((1,) + B2.shape[1:], wmap),
            pl.BlockSpec((1,) + W3.shape[1:], wmap),
            pl.BlockSpec((1,) + B3.shape[1:], wmap),
        ],
        out_specs=pl.BlockSpec((1, B, 1024), wmap),
        out_shape=jax.ShapeDtypeStruct((6, B, 1024), jnp.float32),
    )(chunks, nat, W1, B1, W2, B2, W3, B3)


def _emb_vq_kernel(feat_ref, w0, b0, wm, bm, cb_ref, q_ref, ssq_ref):
    f = feat_ref[0]
    h = jnp.maximum(_bdot(f, w0[0]) + b0[0], 0.0)
    z = _bdot(h, wm[0]) + bm[0]
    cb = cb_ref[0]
    d = (jnp.sum(z * z, axis=1, keepdims=True)
         - 2.0 * _dot_t(z, cb)
         + jnp.sum(cb * cb, axis=1)[None, :])
    B = d.shape[0]
    lane = jax.lax.broadcasted_iota(jnp.int32, (B, 128), 1)
    dmin = jnp.min(d, axis=1, keepdims=True)
    idx = jnp.min(jnp.where(d == dmin, lane, 128), axis=1, keepdims=True)
    onehot = (lane == idx).astype(jnp.float32)
    q = _dot(onehot, cb)
    q_ref[0] = q
    ssq_ref[...] = jnp.sum((q - z) ** 2).reshape(1, 1, 1)


def _emb_vq(feat, emb, cbs):
    B = feat.shape[1]
    stk = lambda k: jnp.stack([emb[i][k] for i in range(6)])
    W0, WM = stk('w0'), stk('wm')
    B0 = stk('b0')[:, None, :]
    BM = stk('bm')[:, None, :]
    CB = jnp.stack(cbs)
    bmap = lambda i: (i, 0, 0)
    return pl.pallas_call(
        _emb_vq_kernel,
        grid=(6,),
        in_specs=[
            pl.BlockSpec((1, B, 1024), bmap),
            pl.BlockSpec((1,) + W0.shape[1:], bmap),
            pl.BlockSpec((1,) + B0.shape[1:], bmap),
            pl.BlockSpec((1,) + WM.shape[1:], bmap),
            pl.BlockSpec((1,) + BM.shape[1:], bmap),
            pl.BlockSpec((1,) + CB.shape[1:], bmap),
        ],
        out_specs=[pl.BlockSpec((1, B, 256), bmap),
                   pl.BlockSpec((1, 1, 1), lambda i: (i, 0, 0))],
        out_shape=[jax.ShapeDtypeStruct((6, B, 256), jnp.float32),
                   jax.ShapeDtypeStruct((6, 1, 1), jnp.float32)],
    )(feat, W0, B0, WM, BM, CB)


def _final_kernel(q_ref, ot_ref, op_ref, cb6_ref, ssq_ref,
                  d0, db0, d1, db1, d2p, db2p,
                  p0, pb0, p1, pb1, p2p, pb2p, out_ref, loss_ref):
    opos = op_ref[...]
    cb6 = cb6_ref[...]
    d = (jnp.sum(opos * opos, axis=1, keepdims=True)
         - 2.0 * _dot_t(opos, cb6)
         + jnp.sum(cb6 * cb6, axis=1)[None, :])
    B = d.shape[0]
    lane = jax.lax.broadcasted_iota(jnp.int32, (B, 128), 1)
    dmin = jnp.min(d, axis=1, keepdims=True)
    idx = jnp.min(jnp.where(d == dmin, lane, 128), axis=1, keepdims=True)
    onehot = (lane == idx).astype(jnp.float32)
    q6 = _dot(onehot, cb6)
    ssq6 = jnp.sum((q6 - opos) ** 2)
    loss = (1.25 * jnp.sum(ssq_ref[...]) / (B * 256.0)
            + 3.0 * ssq6 / (B * 1024.0))

    otype = ot_ref[...]
    x1 = _bdot(otype, d0[1536:, :]) + db0[...]
    for i in range(6):
        x1 = x1 + _bdot(q_ref[i], d0[256 * i:256 * (i + 1), :])
    h = jnp.maximum(x1, 0.0)
    h = jnp.maximum(_bdot(h, d1[...]) + db1[...], 0.0)
    recon = _bdot(h, d2p[...]) + db2p[...]
    y1 = _bdot(q6, p0[:1024, :]) + _bdot(otype, p0[1024:, :]) + pb0[...]
    g = jnp.maximum(y1, 0.0)
    g = jnp.maximum(_bdot(g, p1[...]) + pb1[...], 0.0)
    pos = _bdot(g, p2p[...]) + pb2p[...]
    out_ref[...] = recon + pos
    loss_ref[...] = loss.reshape(1, 1)


def _final(q, otype, opos, cb6, ssq, dec, pos_dec):
    B = otype.shape[0]
    d2p = jnp.pad(dec['w2'], ((0, 0), (0, 6)))
    db2p = jnp.pad(dec['b2'], (0, 6)).reshape(1, 61)
    p2p = jnp.pad(pos_dec['w2'], ((0, 0), (55, 0)))
    pb2p = jnp.pad(pos_dec['b2'], (55, 0)).reshape(1, 61)
    args = [q, otype, opos, cb6, ssq,
            dec['w0'], dec['b0'].reshape(1, -1),
            dec['w1'], dec['b1'].reshape(1, -1), d2p, db2p,
            pos_dec['w0'], pos_dec['b0'].reshape(1, -1),
            pos_dec['w1'], pos_dec['b1'].reshape(1, -1), p2p, pb2p]
    out, loss = pl.pallas_call(
        _final_kernel,
        in_specs=[pl.BlockSpec(a.shape, functools.partial(lambda n: (0,) * n, a.ndim))
                  for a in args],
        out_specs=[pl.BlockSpec((B, 61), lambda: (0, 0)),
                   pl.BlockSpec((1, 1), lambda: (0, 0))],
        out_shape=[jax.ShapeDtypeStruct((B, 61), jnp.float32),
                   jax.ShapeDtypeStruct((1, 1), jnp.float32)],
    )(*args)
    return out, loss


def kernel(obj_pc, hand_xyz, params):
    objx = jnp.transpose(obj_pc, (0, 2, 1))
    otype, opos = _obj_pointnets(objx, params['obj_type'], params['obj_pos'])
    feat = _hand_pointnets(hand_xyz, params['hand_enc'])
    q, ssq = _emb_vq(feat, params['emb'], params['cb'])
    out, loss = _final(q, otype, opos, params['cb6'], ssq,
                       params['dec'], params['pos_dec'])
    return out, loss[0, 0]


# speedup vs baseline: 4.9076x; 1.0845x over previous
import functools

import jax
import jax.numpy as jnp
from jax.experimental import pallas as pl

def _dot(a, b):
    return jnp.dot(a, b, preferred_element_type=jnp.float32)


_bdot = _dot


def _dot_t(a, b):
    return jax.lax.dot_general(a, b, (((1,), (1,)), ((), ())),
                               preferred_element_type=jnp.float32)


_BPC = 2
_NPB = 2048


def _dot0(a, b):
    return jax.lax.dot_general(a, b, (((0,), (0,)), ((), ())),
                               preferred_element_type=jnp.float32)


def _obj_pn_kernel(x_ref, wt1, bt1, wt2, bt2, wt3, bt3,
                   wp1, bp1, wp2, bp2, wp3, bp3, ot_ref, op_ref):
    def chain(xb, w1, b1, w2, b2, w3, b3):
        h = jnp.maximum(_dot0(xb, w1[...]) + b1[...], 0.0)
        h = jnp.maximum(_bdot(h, w2[...]) + b2[...], 0.0)
        h = _bdot(h, w3[...]) + b3[...]
        return jnp.max(h, axis=0, keepdims=True)

    pts, pps = [], []
    for i in range(_BPC):
        xb = x_ref[i]
        pts.append(chain(xb, wt1, bt1, wt2, bt2, wt3, bt3))
        pps.append(chain(xb, wp1, bp1, wp2, bp2, wp3, bp3))
    ot_ref[...] = jnp.stack(pts)
    op_ref[...] = jnp.stack(pps)


def _obj_pointnets(obj_pc, pt_t, pt_p):
    B, C, N = obj_pc.shape
    steps = B // _BPC
    full = lambda s: pl.BlockSpec(s, lambda c: (0,) * len(s))
    wspecs = []
    args = []
    for p in (pt_t, pt_p):
        for k in ('w1', 'b1', 'w2', 'b2', 'w3', 'b3'):
            a = p[k]
            if a.ndim == 1:
                a = a.reshape(1, -1)
            args.append(a)
            wspecs.append(full(a.shape))
    ot, op = pl.pallas_call(
        _obj_pn_kernel,
        grid=(steps,),
        in_specs=[pl.BlockSpec((_BPC, C, N), lambda c: (c, 0, 0))] + wspecs,
        out_specs=[pl.BlockSpec((_BPC, 1, 1024), lambda c: (c, 0, 0)),
                   pl.BlockSpec((_BPC, 1, 1024), lambda c: (c, 0, 0))],
        out_shape=[jax.ShapeDtypeStruct((B, 1, 1024), jnp.float32),
                   jax.ShapeDtypeStruct((B, 1, 1024), jnp.float32)],
    )(obj_pc, *args)
    return ot.reshape(B, 1024), op.reshape(B, 1024)


_FJ = [0, 1, 2, 3, 4, 5, 5]
_STARTS = [0, 83, 206, 326, 448, 569, 697]
_VALID = [83, 123, 120, 122, 121, 128, 81]


def _hand_pn_kernel(chunk_ref, nat_ref, w1, b1, w2, b2, w3, b3, out_ref):
    j = pl.program_id(0)
    v = jnp.int32(_VALID[-1])
    for jj in range(6):
        v = jnp.where(j == jj, jnp.int32(_VALID[jj]), v)
    nat = nat_ref[...]
    mean = jnp.sum(nat, axis=2) * (1.0 / 778.0)
    B = nat.shape[0]
    x = chunk_ref[0].reshape(B, 128, 3) - mean[:, None, :]
    x = x.reshape(B * 128, 3)
    h = jnp.maximum(_bdot(x, w1[0]) + b1[0], 0.0)
    h = jnp.maximum(_bdot(h, w2[0]) + b2[0], 0.0)
    h = _bdot(h, w3[0]) + b3[0]
    h = h.reshape(B, 128, 1024)
    pid = jax.lax.broadcasted_iota(jnp.int32, (B, 128, 1), 1)
    h = jnp.where(pid < v, h, -1e30)
    pm = jnp.max(h, axis=1)

    @pl.when(j < 6)
    def _():
        out_ref[0] = pm

    @pl.when(j == 6)
    def _():
        out_ref[0] = jnp.maximum(out_ref[0], pm)


def _hand_pointnets(hand_xyz, enc):
    B = hand_xyz.shape[0]
    hp = jnp.transpose(hand_xyz, (0, 2, 1))
    hp = jnp.pad(hp, ((0, 0), (0, 832 - 778), (0, 0)))
    chunks = jnp.stack([hp[:, s:s + 128, :] for s in _STARTS])
    chunks = chunks.reshape(7, B * 128, 3)
    nat = jnp.pad(hand_xyz, ((0, 0), (0, 0), (0, 832 - 778)))

    stk = lambda k: jnp.stack([enc[i][k] for i in range(6)])
    W1, W2, W3 = stk('w1'), stk('w2'), stk('w3')
    B1 = stk('b1')[:, None, :]
    B2 = stk('b2')[:, None, :]
    B3 = stk('b3')[:, None, :]

    wmap = lambda j: (jnp.minimum(j, 5), 0, 0)
    return pl.pallas_call(
        _hand_pn_kernel,
        grid=(7,),
        in_specs=[
            pl.BlockSpec((1, B * 128, 3), lambda j: (j, 0, 0)),
            pl.BlockSpec(nat.shape, lambda j: (0, 0, 0)),
            pl.BlockSpec((1,) + W1.shape[1:], wmap),
            pl.BlockSpec((1,) + B1.shape[1:], wmap),
            pl.BlockSpec((1,) + W2.shape[1:], wmap),
            pl.BlockSpec((1,) + B2.shape[1:], wmap),
            pl.BlockSpec((1,) + W3.shape[1:], wmap),
            pl.BlockSpec((1,) + B3.shape[1:], wmap),
        ],
        out_specs=pl.BlockSpec((1, B, 1024), wmap),
        out_shape=jax.ShapeDtypeStruct((6, B, 1024), jnp.float32),
    )(chunks, nat, W1, B1, W2, B2, W3, B3)


def _emb_vq_kernel(feat_ref, w0, b0, wm, bm, cb_ref, q_ref, ssq_ref):
    f = feat_ref[0]
    h = jnp.maximum(_bdot(f, w0[0]) + b0[0], 0.0)
    z = _bdot(h, wm[0]) + bm[0]
    cb = cb_ref[0]
    d = (jnp.sum(z * z, axis=1, keepdims=True)
         - 2.0 * _dot_t(z, cb)
         + jnp.sum(cb * cb, axis=1)[None, :])
    B = d.shape[0]
    lane = jax.lax.broadcasted_iota(jnp.int32, (B, 128), 1)
    dmin = jnp.min(d, axis=1, keepdims=True)
    idx = jnp.min(jnp.where(d == dmin, lane, 128), axis=1, keepdims=True)
    onehot = (lane == idx).astype(jnp.float32)
    q = _dot(onehot, cb)
    q_ref[0] = q
    ssq_ref[...] = jnp.sum((q - z) ** 2).reshape(1, 1, 1)


def _emb_vq(feat, emb, cbs):
    B = feat.shape[1]
    stk = lambda k: jnp.stack([emb[i][k] for i in range(6)])
    W0, WM = stk('w0'), stk('wm')
    B0 = stk('b0')[:, None, :]
    BM = stk('bm')[:, None, :]
    CB = jnp.stack(cbs)
    bmap = lambda i: (i, 0, 0)
    return pl.pallas_call(
        _emb_vq_kernel,
        grid=(6,),
        in_specs=[
            pl.BlockSpec((1, B, 1024), bmap),
            pl.BlockSpec((1,) + W0.shape[1:], bmap),
            pl.BlockSpec((1,) + B0.shape[1:], bmap),
            pl.BlockSpec((1,) + WM.shape[1:], bmap),
            pl.BlockSpec((1,) + BM.shape[1:], bmap),
            pl.BlockSpec((1,) + CB.shape[1:], bmap),
        ],
        out_specs=[pl.BlockSpec((1, B, 256), bmap),
                   pl.BlockSpec((1, 1, 1), lambda i: (i, 0, 0))],
        out_shape=[jax.ShapeDtypeStruct((6, B, 256), jnp.float32),
                   jax.ShapeDtypeStruct((6, 1, 1), jnp.float32)],
    )(feat, W0, B0, WM, BM, CB)


def _final_kernel(q_ref, ot_ref, op_ref, cb6_ref, ssq_ref,
                  d0, db0, d1, db1, d2p, db2p,
                  p0, pb0, p1, pb1, p2p, pb2p, out_ref, loss_ref):
    opos = op_ref[...]
    cb6 = cb6_ref[...]
    d = (jnp.sum(opos * opos, axis=1, keepdims=True)
         - 2.0 * _dot_t(opos, cb6)
         + jnp.sum(cb6 * cb6, axis=1)[None, :])
    B = d.shape[0]
    lane = jax.lax.broadcasted_iota(jnp.int32, (B, 128), 1)
    dmin = jnp.min(d, axis=1, keepdims=True)
    idx = jnp.min(jnp.where(d == dmin, lane, 128), axis=1, keepdims=True)
    onehot = (lane == idx).astype(jnp.float32)
    q6 = _dot(onehot, cb6)
    ssq6 = jnp.sum((q6 - opos) ** 2)
    loss = (1.25 * jnp.sum(ssq_ref[...]) / (B * 256.0)
            + 3.0 * ssq6 / (B * 1024.0))

    otype = ot_ref[...]
    x1 = _bdot(otype, d0[1536:, :]) + db0[...]
    for i in range(6):
        x1 = x1 + _bdot(q_ref[i], d0[256 * i:256 * (i + 1), :])
    h = jnp.maximum(x1, 0.0)
    h = jnp.maximum(_bdot(h, d1[...]) + db1[...], 0.0)
    recon = _bdot(h, d2p[...]) + db2p[...]
    y1 = _bdot(q6, p0[:1024, :]) + _bdot(otype, p0[1024:, :]) + pb0[...]
    g = jnp.maximum(y1, 0.0)
    g = jnp.maximum(_bdot(g, p1[...]) + pb1[...], 0.0)
    pos = _bdot(g, p2p[...]) + pb2p[...]
    out_ref[...] = recon + pos
    loss_ref[...] = loss.reshape(1, 1)


def _final(q, otype, opos, cb6, ssq, dec, pos_dec):
    B = otype.shape[0]
    d2p = jnp.pad(dec['w2'], ((0, 0), (0, 6)))
    db2p = jnp.pad(dec['b2'], (0, 6)).reshape(1, 61)
    p2p = jnp.pad(pos_dec['w2'], ((0, 0), (55, 0)))
    pb2p = jnp.pad(pos_dec['b2'], (55, 0)).reshape(1, 61)
    args = [q, otype, opos, cb6, ssq,
            dec['w0'], dec['b0'].reshape(1, -1),
            dec['w1'], dec['b1'].reshape(1, -1), d2p, db2p,
            pos_dec['w0'], pos_dec['b0'].reshape(1, -1),
            pos_dec['w1'], pos_dec['b1'].reshape(1, -1), p2p, pb2p]
    out, loss = pl.pallas_call(
        _final_kernel,
        in_specs=[pl.BlockSpec(a.shape, functools.partial(lambda n: (0,) * n, a.ndim))
                  for a in args],
        out_specs=[pl.BlockSpec((B, 61), lambda: (0, 0)),
                   pl.BlockSpec((1, 1), lambda: (0, 0))],
        out_shape=[jax.ShapeDtypeStruct((B, 61), jnp.float32),
                   jax.ShapeDtypeStruct((1, 1), jnp.float32)],
    )(*args)
    return out, loss


def kernel(obj_pc, hand_xyz, params):
    otype, opos = _obj_pointnets(obj_pc, params['obj_type'], params['obj_pos'])
    feat = _hand_pointnets(hand_xyz, params['hand_enc'])
    q, ssq = _emb_vq(feat, params['emb'], params['cb'])
    out, loss = _final(q, otype, opos, params['cb6'], ssq,
                       params['dec'], params['pos_dec'])
    return out, loss[0, 0]
